# Initial kernel scaffold; baseline (speedup 1.0000x reference)
import jax
import jax.numpy as jnp
from jax.experimental import pallas as pl


def kernel(idx, targets, table):
    raise NotImplementedError("write your pallas kernel here")


# speedup vs baseline: 1.3772x; 1.3772x over previous
import jax
import jax.numpy as jnp
from jax import lax
from jax.experimental import pallas as pl
from jax.experimental.pallas import tpu as pltpu
from jax.experimental.pallas import tpu_sc as plsc
import functools

NC = 2
NS = 16
NW = NC * NS
LANES = 16

V = 1000
BT = 4096 * 200
RPW = BT // NW
CHUNK = 32
NCH = RPW // CHUNK


def _lse_body(tbl_ref, out_ref):
    x = tbl_ref[...]
    m = jnp.max(x, axis=1, keepdims=True)
    s = jnp.sum(jnp.exp(x - m), axis=1, keepdims=True)
    out_ref[...] = m + jnp.log(s)


_lse_call = pl.pallas_call(
    _lse_body,
    out_shape=jax.ShapeDtypeStruct((V, 1), jnp.float32),
)


def _sc_body(table_hbm, idx_hbm, tgt_hbm, lse_hbm, out_hbm, part_hbm,
             idx_v, tgt_v, lse_v, acc_v, rows0, rows1,
             gsem0, gsem1, osem0, osem1):
    wid = lax.axis_index("s") * NC + lax.axis_index("c")
    base = wid * RPW

    pltpu.sync_copy(idx_hbm.at[pl.ds(base, RPW)], idx_v)
    pltpu.sync_copy(tgt_hbm.at[pl.ds(base, RPW)], tgt_v)
    pltpu.sync_copy(lse_hbm, lse_v)
    acc_v[...] = jnp.zeros((LANES,), jnp.float32)

    rows = (rows0, rows1)
    gsems = (gsem0, gsem1)
    osems = (osem0, osem1)

    def gather_desc(g, b):
        return pltpu.make_async_copy(
            table_hbm.at[idx_v.at[pl.ds(g * CHUNK, CHUNK)]], rows[b], gsems[b])

    def write_desc(g, b):
        return pltpu.make_async_copy(
            rows[b], out_hbm.at[pl.ds(base + g * CHUNK, CHUNK)], osems[b])

    def extract(g, b):
        rb = rows[b]
        for k in range(CHUNK // LANES):
            off = g * CHUNK + k * LANES
            tg = tgt_v[pl.ds(off, LANES)]
            ix = idx_v[pl.ds(off, LANES)]
            rowid = lax.iota(jnp.int32, LANES) + (k * LANES)
            vals = plsc.load_gather(rb, [rowid, tg])
            lses = plsc.load_gather(lse_v, [ix])
            acc_v[...] = acc_v[...] + (lses - vals)

    gather_desc(0, 0).start()

    def outer(i, carry):
        g0 = i * 2
        gather_desc(g0, 0).wait()
        extract(g0, 0)
        write_desc(g0, 0).start()

        @pl.when(g0 > 0)
        def _():
            write_desc(g0 - 1, 1).wait()

        gather_desc(g0 + 1, 1).start()

        gather_desc(g0 + 1, 1).wait()
        extract(g0 + 1, 1)
        write_desc(g0 + 1, 1).start()
        write_desc(g0, 0).wait()

        @pl.when(g0 + 2 < NCH)
        def _():
            gather_desc(g0 + 2, 0).start()

        return carry

    lax.fori_loop(0, NCH // 2, outer, 0)
    write_desc(NCH - 1, 1).wait()
    pltpu.sync_copy(acc_v, part_hbm.at[wid])


_sc_call = pl.kernel(
    _sc_body,
    out_type=(
        jax.ShapeDtypeStruct((BT, V), jnp.float32),
        jax.ShapeDtypeStruct((NW, LANES), jnp.float32),
    ),
    mesh=plsc.VectorSubcoreMesh(core_axis_name="c", subcore_axis_name="s",
                                num_cores=NC, num_subcores=NS),
    scratch_types=[
        pltpu.VMEM((RPW,), jnp.int32),
        pltpu.VMEM((RPW,), jnp.int32),
        pltpu.VMEM((V,), jnp.float32),
        pltpu.VMEM((LANES,), jnp.float32),
        pltpu.VMEM((CHUNK, V), jnp.float32),
        pltpu.VMEM((CHUNK, V), jnp.float32),
        pltpu.SemaphoreType.DMA,
        pltpu.SemaphoreType.DMA,
        pltpu.SemaphoreType.DMA,
        pltpu.SemaphoreType.DMA,
    ],
    compiler_params=pltpu.CompilerParams(use_tc_tiling_on_sc=False,
                                         needs_layout_passes=False),
)


@jax.jit
def kernel(idx, targets, table):
    Bb, Tt = idx.shape
    lse = _lse_call(table).reshape(V)
    logits_flat, partials = _sc_call(
        table, idx.reshape(-1), targets.reshape(-1), lse)
    loss = jnp.sum(partials) / (Bb * Tt)
    return logits_flat.reshape(Bb, Tt, V), loss


# speedup vs baseline: 1.9844x; 1.4409x over previous
import jax
import jax.numpy as jnp
from jax import lax
from jax.experimental import pallas as pl
from jax.experimental.pallas import tpu as pltpu
from jax.experimental.pallas import tpu_sc as plsc
import functools

NC = 2
NS = 16
NW = NC * NS
LANES = 16

V = 1000
VP = 1024
B, T = 4096, 200
BT = B * T
K = 4
TCH = T // K
QCH = TCH * B
RPW = QCH // NW
CHUNK = 32
NCH = RPW // CHUNK
BB = 256


def _lse_body(tbl_ref, out_ref):
    x = tbl_ref[...]
    m = jnp.max(x, axis=1, keepdims=True)
    s = jnp.sum(jnp.exp(x - m), axis=1, keepdims=True)
    out_ref[...] = m + jnp.log(s)


_lse_call = pl.pallas_call(
    _lse_body,
    out_shape=jax.ShapeDtypeStruct((V, 1), jnp.float32),
)


def _sc_body(table_hbm, idx_hbm, tgt_hbm, lse_hbm, out_hbm, part_hbm,
             idx_v, tgt_v, lse_v, acc_v, rows0, rows1,
             gsem0, gsem1, osem0, osem1):
    wid = lax.axis_index("s") * NC + lax.axis_index("c")
    base = wid * RPW

    pltpu.sync_copy(idx_hbm.at[pl.ds(base, RPW)], idx_v)
    pltpu.sync_copy(tgt_hbm.at[pl.ds(base, RPW)], tgt_v)
    pltpu.sync_copy(lse_hbm, lse_v)
    acc_v[...] = jnp.zeros((LANES,), jnp.float32)

    rows = (rows0, rows1)
    gsems = (gsem0, gsem1)
    osems = (osem0, osem1)

    def gather_desc(g, b):
        return pltpu.make_async_copy(
            table_hbm.at[idx_v.at[pl.ds(g * CHUNK, CHUNK)]], rows[b], gsems[b])

    def write_desc(g, b):
        return pltpu.make_async_copy(
            rows[b], out_hbm.at[pl.ds(base + g * CHUNK, CHUNK), pl.ds(0, V)],
            osems[b])

    def extract(g, b):
        rb = rows[b]
        for k in range(CHUNK // LANES):
            off = g * CHUNK + k * LANES
            tg = tgt_v[pl.ds(off, LANES)]
            ix = idx_v[pl.ds(off, LANES)]
            rowid = lax.iota(jnp.int32, LANES) + (k * LANES)
            vals = plsc.load_gather(rb, [rowid, tg])
            lses = plsc.load_gather(lse_v, [ix])
            acc_v[...] = acc_v[...] + (lses - vals)

    gather_desc(0, 0).start()

    def outer(i, carry):
        g0 = i * 2
        gather_desc(g0, 0).wait()
        extract(g0, 0)
        write_desc(g0, 0).start()

        @pl.when(g0 > 0)
        def _():
            write_desc(g0 - 1, 1).wait()

        gather_desc(g0 + 1, 1).start()

        gather_desc(g0 + 1, 1).wait()
        extract(g0 + 1, 1)
        write_desc(g0 + 1, 1).start()
        write_desc(g0, 0).wait()

        @pl.when(g0 + 2 < NCH)
        def _():
            gather_desc(g0 + 2, 0).start()

        return carry

    lax.fori_loop(0, NCH // 2, outer, 0)
    write_desc(NCH - 1, 1).wait()
    pltpu.sync_copy(acc_v, part_hbm.at[wid])


_sc_call = pl.kernel(
    _sc_body,
    out_type=(
        jax.ShapeDtypeStruct((QCH, VP), jnp.float32),
        jax.ShapeDtypeStruct((NW, LANES), jnp.float32),
    ),
    mesh=plsc.VectorSubcoreMesh(core_axis_name="c", subcore_axis_name="s",
                                num_cores=NC, num_subcores=NS),
    scratch_types=[
        pltpu.VMEM((RPW,), jnp.int32),
        pltpu.VMEM((RPW,), jnp.int32),
        pltpu.VMEM((V,), jnp.float32),
        pltpu.VMEM((LANES,), jnp.float32),
        pltpu.VMEM((CHUNK, V), jnp.float32),
        pltpu.VMEM((CHUNK, V), jnp.float32),
        pltpu.SemaphoreType.DMA,
        pltpu.SemaphoreType.DMA,
        pltpu.SemaphoreType.DMA,
        pltpu.SemaphoreType.DMA,
    ],
    compiler_params=pltpu.CompilerParams(use_tc_tiling_on_sc=False,
                                         needs_layout_passes=False),
)


def _tr_body(k, in_ref, prev_ref, out_ref):
    buf = in_ref[...]
    for s in range(7):
        out_ref[0, pl.ds(s * 128, 128), :] = jnp.transpose(buf[:, s, :], (1, 0))
    out_ref[0, pl.ds(896, V - 896), :] = (
        jnp.transpose(buf[:, 7, :], (1, 0))[: V - 896, :])


def _make_tr_call(k, aliased):
    kwargs = {}
    in_specs = [pl.BlockSpec((BB, 8, 128), lambda t, bt: (t * (B // BB) + bt, 0, 0))]
    if aliased:
        in_specs.append(pl.BlockSpec(memory_space=pl.ANY))
        kwargs["input_output_aliases"] = {1: 0}

        def body(in_ref, prev_ref, out_ref):
            _tr_body(k, in_ref, prev_ref, out_ref)
    else:
        def body(in_ref, out_ref):
            _tr_body(k, in_ref, None, out_ref)
    return pl.pallas_call(
        body,
        grid=(TCH, B // BB),
        in_specs=in_specs,
        out_specs=pl.BlockSpec((1, V, BB), lambda t, bt: (k * TCH + t, 0, bt)),
        out_shape=jax.ShapeDtypeStruct((T, V, B), jnp.float32),
        **kwargs,
    )


_tr_calls = [_make_tr_call(k, aliased=(k > 0)) for k in range(K)]


@jax.jit
def kernel(idx, targets, table):
    lse = _lse_call(table).reshape(V)
    idx_t = jnp.transpose(idx).reshape(-1)
    tgt_t = jnp.transpose(targets).reshape(-1)

    parts = []
    out = None
    for k in range(K):
        inter, part = _sc_call(
            table,
            lax.slice(idx_t, (k * QCH,), ((k + 1) * QCH,)),
            lax.slice(tgt_t, (k * QCH,), ((k + 1) * QCH,)),
            lse,
        )
        parts.append(part)
        inter3 = inter.reshape(QCH, 8, 128)
        if k == 0:
            out = _tr_calls[0](inter3)
        else:
            out = _tr_calls[k](inter3, out)

    loss = jnp.sum(jnp.stack(parts)) / BT
    return jnp.transpose(out, (2, 0, 1)), loss


# speedup vs baseline: 2.4876x; 1.2536x over previous
import jax
import jax.numpy as jnp
from jax import lax
from jax.experimental import pallas as pl
from jax.experimental.pallas import tpu as pltpu
from jax.experimental.pallas import tpu_sc as plsc
import functools

NC = 2
NS = 16
NW = NC * NS
LANES = 16

V = 1000
VP = 1024
B, T = 4096, 200
BT = B * T
K = 4
TCH = T // K
QCH = TCH * B
RPW = QCH // NW
CHUNK = 32
PH = 4
RPP = RPW // PH
NCHP = RPP // CHUNK
BB = 256


def _lse_body(tbl_ref, out_ref):
    x = tbl_ref[...]
    m = jnp.max(x, axis=1, keepdims=True)
    s = jnp.sum(jnp.exp(x - m), axis=1, keepdims=True)
    out_ref[...] = m + jnp.log(s)


_lse_call = pl.pallas_call(
    _lse_body,
    out_shape=jax.ShapeDtypeStruct((V, 1), jnp.float32),
)


def _sc_body(table_hbm, idx_hbm, tgt_hbm, lse_hbm, out_hbm, part_hbm,
             tsh, idx_v, tgt_v, lse_v, acc_v, rows0, rows1,
             gsem0, gsem1, osem0, osem1):
    sid = lax.axis_index("s")
    wid = sid * NC + lax.axis_index("c")
    base = wid * RPW

    @pl.when(sid == 0)
    def _():
        pltpu.sync_copy(table_hbm, tsh)

    pltpu.sync_copy(lse_hbm, lse_v)
    acc_v[...] = jnp.zeros((LANES,), jnp.float32)
    plsc.subcore_barrier()

    rows = (rows0, rows1)
    gsems = (gsem0, gsem1)
    osems = (osem0, osem1)

    for ph in range(PH):
        pbase = base + ph * RPP
        pltpu.sync_copy(idx_hbm.at[pl.ds(pbase, RPP)], idx_v)
        pltpu.sync_copy(tgt_hbm.at[pl.ds(pbase, RPP)], tgt_v)

        def gather_desc(g, b):
            return pltpu.make_async_copy(
                tsh.at[idx_v.at[pl.ds(g * CHUNK, CHUNK)]], rows[b], gsems[b])

        def write_desc(g, b, pbase=pbase):
            return pltpu.make_async_copy(
                rows[b],
                out_hbm.at[pl.ds(pbase + g * CHUNK, CHUNK), pl.ds(0, V)],
                osems[b])

        def extract(g, b):
            rb = rows[b]
            for k in range(CHUNK // LANES):
                off = g * CHUNK + k * LANES
                tg = tgt_v[pl.ds(off, LANES)]
                ix = idx_v[pl.ds(off, LANES)]
                rowid = lax.iota(jnp.int32, LANES) + (k * LANES)
                vals = plsc.load_gather(rb, [rowid, tg])
                lses = plsc.load_gather(lse_v, [ix])
                acc_v[...] = acc_v[...] + (lses - vals)

        gather_desc(0, 0).start()

        def outer(i, carry):
            g0 = i * 2
            gather_desc(g0, 0).wait()
            extract(g0, 0)
            write_desc(g0, 0).start()

            @pl.when(g0 > 0)
            def _():
                write_desc(g0 - 1, 1).wait()

            gather_desc(g0 + 1, 1).start()

            gather_desc(g0 + 1, 1).wait()
            extract(g0 + 1, 1)
            write_desc(g0 + 1, 1).start()
            write_desc(g0, 0).wait()

            @pl.when(g0 + 2 < NCHP)
            def _():
                gather_desc(g0 + 2, 0).start()

            return carry

        lax.fori_loop(0, NCHP // 2, outer, 0)
        write_desc(NCHP - 1, 1).wait()

    pltpu.sync_copy(acc_v, part_hbm.at[wid])


_sc_call = pl.kernel(
    _sc_body,
    out_type=(
        jax.ShapeDtypeStruct((QCH, VP), jnp.float32),
        jax.ShapeDtypeStruct((NW, LANES), jnp.float32),
    ),
    mesh=plsc.VectorSubcoreMesh(core_axis_name="c", subcore_axis_name="s",
                                num_cores=NC, num_subcores=NS),
    scratch_types=[
        pltpu.VMEM_SHARED((V, V), jnp.float32),
        pltpu.VMEM((RPP,), jnp.int32),
        pltpu.VMEM((RPP,), jnp.int32),
        pltpu.VMEM((V,), jnp.float32),
        pltpu.VMEM((LANES,), jnp.float32),
        pltpu.VMEM((CHUNK, V), jnp.float32),
        pltpu.VMEM((CHUNK, V), jnp.float32),
        pltpu.SemaphoreType.DMA,
        pltpu.SemaphoreType.DMA,
        pltpu.SemaphoreType.DMA,
        pltpu.SemaphoreType.DMA,
    ],
    compiler_params=pltpu.CompilerParams(use_tc_tiling_on_sc=False,
                                         needs_layout_passes=False),
)


def _tr_body(k, in_ref, prev_ref, out_ref):
    buf = in_ref[...]
    for s in range(7):
        out_ref[0, pl.ds(s * 128, 128), :] = jnp.transpose(buf[:, s, :], (1, 0))
    out_ref[0, pl.ds(896, V - 896), :] = (
        jnp.transpose(buf[:, 7, :], (1, 0))[: V - 896, :])


def _make_tr_call(k, aliased):
    kwargs = {}
    in_specs = [pl.BlockSpec((BB, 8, 128), lambda t, bt: (t * (B // BB) + bt, 0, 0))]
    if aliased:
        in_specs.append(pl.BlockSpec(memory_space=pl.ANY))
        kwargs["input_output_aliases"] = {1: 0}

        def body(in_ref, prev_ref, out_ref):
            _tr_body(k, in_ref, prev_ref, out_ref)
    else:
        def body(in_ref, out_ref):
            _tr_body(k, in_ref, None, out_ref)
    return pl.pallas_call(
        body,
        grid=(TCH, B // BB),
        in_specs=in_specs,
        out_specs=pl.BlockSpec((1, V, BB), lambda t, bt: (k * TCH + t, 0, bt)),
        out_shape=jax.ShapeDtypeStruct((T, V, B), jnp.float32),
        **kwargs,
    )


_tr_calls = [_make_tr_call(k, aliased=(k > 0)) for k in range(K)]


@jax.jit
def kernel(idx, targets, table):
    lse = _lse_call(table).reshape(V)
    idx_t = jnp.transpose(idx).reshape(-1)
    tgt_t = jnp.transpose(targets).reshape(-1)

    parts = []
    out = None
    for k in range(K):
        inter, part = _sc_call(
            table,
            lax.slice(idx_t, (k * QCH,), ((k + 1) * QCH,)),
            lax.slice(tgt_t, (k * QCH,), ((k + 1) * QCH,)),
            lse,
        )
        parts.append(part)
        inter3 = inter.reshape(QCH, 8, 128)
        if k == 0:
            out = _tr_calls[0](inter3)
        else:
            out = _tr_calls[k](inter3, out)

    loss = jnp.sum(jnp.stack(parts)) / BT
    return jnp.transpose(out, (2, 0, 1)), loss
